# baseline (speedup 1.0000x reference)
import functools

import jax
import jax.numpy as jnp
from jax import lax
from jax.experimental import pallas as pl
from jax.experimental.pallas import tpu as pltpu
from jax.experimental.pallas import tpu_sc as plsc

B = 16384
DIM = 64
L = 16
NC = 2
NS = 16
NW = NC * NS
BPW = B // NW
NG = BPW // L
L2 = 1e-4


def _sc_body(users_hbm, pos_hbm, neg_hbm, uemb_hbm, iemb_hbm,
             pos_out, neg_out, reg_out,
             idx_u, idx_p, idx_n, rows_u, rows_p, rows_n,
             pos_buf, neg_buf, reg_buf, sem):
    wid = lax.axis_index("s") * NC + lax.axis_index("c")
    base = wid * BPW

    pltpu.sync_copy(users_hbm.at[pl.ds(base, BPW)], idx_u)
    pltpu.sync_copy(pos_hbm.at[pl.ds(base, BPW)], idx_p)
    pltpu.sync_copy(neg_hbm.at[pl.ds(base, BPW)], idx_n)

    cp_u = pltpu.async_copy(uemb_hbm.at[idx_u], rows_u, sem)
    cp_p = pltpu.async_copy(iemb_hbm.at[idx_p], rows_p, sem)
    cp_n = pltpu.async_copy(iemb_hbm.at[idx_n], rows_n, sem)
    cp_u.wait()
    cp_p.wait()
    cp_n.wait()

    iota = lax.iota(jnp.int32, L)

    def group(g, acc_sq):
        rows = g * L + iota
        acc_p = jnp.zeros((L,), jnp.float32)
        acc_n = jnp.zeros((L,), jnp.float32)
        for d in range(DIM):
            col = jnp.full((L,), d, jnp.int32)
            gu = plsc.load_gather(rows_u, [rows, col])
            gp = plsc.load_gather(rows_p, [rows, col])
            gn = plsc.load_gather(rows_n, [rows, col])
            acc_p = acc_p + gu * gp
            acc_n = acc_n + gu * gn
            acc_sq = acc_sq + gu * gu
            acc_sq = acc_sq + gp * gp
            acc_sq = acc_sq + gn * gn
        pos_buf[pl.ds(g * L, L)] = acc_p
        neg_buf[pl.ds(g * L, L)] = acc_n
        return acc_sq

    acc_sq = lax.fori_loop(0, NG, group, jnp.zeros((L,), jnp.float32))
    reg_buf[...] = acc_sq

    pltpu.sync_copy(pos_buf, pos_out.at[pl.ds(base, BPW)])
    pltpu.sync_copy(neg_buf, neg_out.at[pl.ds(base, BPW)])
    pltpu.sync_copy(reg_buf, reg_out.at[wid])


@functools.partial(jax.jit, static_argnums=())
def _sc_gather_scores(users, pos_items, neg0, user_emb, item_emb):
    mesh = plsc.VectorSubcoreMesh(core_axis_name="c", subcore_axis_name="s",
                                  num_cores=NC, num_subcores=NS)
    f = pl.kernel(
        _sc_body,
        out_type=[
            jax.ShapeDtypeStruct((B,), jnp.float32),
            jax.ShapeDtypeStruct((B,), jnp.float32),
            jax.ShapeDtypeStruct((NW, L), jnp.float32),
        ],
        mesh=mesh,
        compiler_params=pltpu.CompilerParams(needs_layout_passes=False,
                                             use_tc_tiling_on_sc=False),
        scratch_types=[
            pltpu.VMEM((BPW,), jnp.int32),
            pltpu.VMEM((BPW,), jnp.int32),
            pltpu.VMEM((BPW,), jnp.int32),
            pltpu.VMEM((BPW, DIM), jnp.float32),
            pltpu.VMEM((BPW, DIM), jnp.float32),
            pltpu.VMEM((BPW, DIM), jnp.float32),
            pltpu.VMEM((BPW,), jnp.float32),
            pltpu.VMEM((BPW,), jnp.float32),
            pltpu.VMEM((L,), jnp.float32),
            pltpu.SemaphoreType.DMA,
        ],
    )
    return f(users, pos_items, neg0, user_emb, item_emb)


def _tc_loss_body(pos_ref, neg_ref, reg_ref, out_ref):
    x = neg_ref[...] - pos_ref[...]
    mf = jnp.sum(jnp.log(1.0 + jnp.exp(x))) / B
    reg = jnp.sum(reg_ref[...])
    out_ref[0, 0] = mf + L2 * reg / (2.0 * B)


def _tc_loss(pos2d, neg2d, reg2d):
    return pl.pallas_call(
        _tc_loss_body,
        out_shape=jax.ShapeDtypeStruct((1, 1), jnp.float32),
        out_specs=pl.BlockSpec(memory_space=pltpu.SMEM),
    )(pos2d, neg2d, reg2d)


def kernel(cur_epoch, users, pos_items, neg_items, user_emb, item_emb):
    users = users.astype(jnp.int32)
    pos_items = pos_items.astype(jnp.int32)
    neg0 = neg_items[:, 0].astype(jnp.int32)
    pos_scores, neg_scores, reg = _sc_gather_scores(
        users, pos_items, neg0, user_emb, item_emb)
    loss = _tc_loss(pos_scores.reshape(128, 128),
                    neg_scores.reshape(128, 128),
                    reg.reshape(4, 128))[0, 0]
    return (loss, pos_scores, neg_scores.reshape(B, 1))
